# baseline (speedup 1.0000x reference)
import jax
import jax.numpy as jnp
from jax.experimental import pallas as pl

_NUM_EXPERTS = 8
_RANK = 16
_SCALING = 2.0
_ER = _NUM_EXPERTS * _RANK
_TB = 1024


def _fused_body(role_ref, x_ref, wat_ref, b_ref, ball_ref, o_ref):
    xb = x_ref[...].astype(jnp.bfloat16)
    r = jnp.dot(xb, wat_ref[...], preferred_element_type=jnp.float32)
    base = r[:, : r.shape[1] - _ER]
    u = r[:, r.shape[1] - _ER :]
    role = role_ref[0, 0, :]
    col_expert = jax.lax.broadcasted_iota(jnp.int32, (1, _ER), 1) // _RANK
    mask = (role[:, None] == col_expert).astype(jnp.float32)
    um = (u * mask).astype(jnp.bfloat16)
    lora = jnp.dot(um, ball_ref[...], preferred_element_type=jnp.float32)
    o_ref[...] = base + _SCALING * lora + b_ref[...]


def kernel(x, role_ids, W, b, A, Bm):
    Bsz, T, D = x.shape
    O = W.shape[0]
    N = Bsz * T
    G = N // _TB
    x_flat = x.reshape(N, D)
    role3 = role_ids.reshape(G, 1, _TB).astype(jnp.int32)
    wt = W.T.astype(jnp.bfloat16)
    at = A.reshape(_ER, D).T.astype(jnp.bfloat16)
    wat = jnp.concatenate([wt, at], axis=1)
    ball = Bm.transpose(0, 2, 1).reshape(_ER, O).astype(jnp.bfloat16)
    b2 = b.reshape(1, O)
    out = pl.pallas_call(
        _fused_body,
        grid=(G,),
        in_specs=[
            pl.BlockSpec((1, 1, _TB), lambda i: (i, 0, 0)),
            pl.BlockSpec((_TB, D), lambda i: (i, 0)),
            pl.BlockSpec((D, O + _ER), lambda i: (0, 0)),
            pl.BlockSpec((1, O), lambda i: (0, 0)),
            pl.BlockSpec((_ER, O), lambda i: (0, 0)),
        ],
        out_specs=pl.BlockSpec((_TB, O), lambda i: (i, 0)),
        out_shape=jax.ShapeDtypeStruct((N, O), jnp.float32),
    )(role3, x_flat, wat, b2, ball)
    return out.reshape(Bsz, T, O)


# speedup vs baseline: 1.1400x; 1.1400x over previous
import jax
import jax.numpy as jnp
from jax.experimental import pallas as pl

_NUM_EXPERTS = 8
_RANK = 16
_SCALING = 2.0
_ER = _NUM_EXPERTS * _RANK
_TB = 512


def _fused_body(role_ref, x_ref, at_ref, wb_ref, b_ref, o_ref):
    xb = x_ref[...].astype(jnp.bfloat16)
    u = jnp.dot(xb, at_ref[...], preferred_element_type=jnp.float32)
    role = role_ref[0, 0, :]
    col_expert = jax.lax.broadcasted_iota(jnp.int32, (1, _ER), 1) // _RANK
    um = jnp.where(role[:, None] == col_expert, u, 0.0).astype(jnp.bfloat16)
    x_cat = jnp.concatenate([xb, um], axis=1)
    o_ref[...] = (
        jnp.dot(x_cat, wb_ref[...], preferred_element_type=jnp.float32)
        + b_ref[...]
    )


def kernel(x, role_ids, W, b, A, Bm):
    Bsz, T, D = x.shape
    O = W.shape[0]
    N = Bsz * T
    G = N // _TB
    x_flat = x.reshape(N, D)
    role3 = role_ids.reshape(G, 1, _TB).astype(jnp.int32)
    at = A.reshape(_ER, D).T.astype(jnp.bfloat16)
    wt = W.T.astype(jnp.bfloat16)
    ball = (Bm.transpose(0, 2, 1).reshape(_ER, O) * _SCALING).astype(jnp.bfloat16)
    wb = jnp.concatenate([wt, ball], axis=0)
    b2 = b.reshape(1, O)
    out = pl.pallas_call(
        _fused_body,
        grid=(G,),
        in_specs=[
            pl.BlockSpec((1, 1, _TB), lambda i: (i, 0, 0)),
            pl.BlockSpec((_TB, D), lambda i: (i, 0)),
            pl.BlockSpec((D, _ER), lambda i: (0, 0)),
            pl.BlockSpec((D + _ER, O), lambda i: (0, 0)),
            pl.BlockSpec((1, O), lambda i: (0, 0)),
        ],
        out_specs=pl.BlockSpec((_TB, O), lambda i: (i, 0)),
        out_shape=jax.ShapeDtypeStruct((N, O), jnp.float32),
    )(role3, x_flat, at, wb, b2)
    return out.reshape(Bsz, T, O)


# speedup vs baseline: 1.1471x; 1.0062x over previous
import jax
import jax.numpy as jnp
from jax.experimental import pallas as pl

_NUM_EXPERTS = 8
_RANK = 16
_SCALING = 2.0
_ER = _NUM_EXPERTS * _RANK
_TB = 1024


def _fused_body(role_ref, x_ref, at_ref, wb_ref, b_ref, o_ref):
    xb = x_ref[...].astype(jnp.bfloat16)
    u = jnp.dot(xb, at_ref[...], preferred_element_type=jnp.float32)
    role = role_ref[0, 0, :]
    col_expert = jax.lax.broadcasted_iota(jnp.int32, (1, _ER), 1) // _RANK
    um = jnp.where(role[:, None] == col_expert, u, 0.0).astype(jnp.bfloat16)
    x_cat = jnp.concatenate([xb, um], axis=1)
    o_ref[...] = (
        jnp.dot(x_cat, wb_ref[...], preferred_element_type=jnp.float32)
        + b_ref[...]
    )


def kernel(x, role_ids, W, b, A, Bm):
    Bsz, T, D = x.shape
    O = W.shape[0]
    N = Bsz * T
    G = N // _TB
    x_flat = x.reshape(N, D)
    role3 = role_ids.reshape(G, 1, _TB).astype(jnp.int32)
    at = A.reshape(_ER, D).T.astype(jnp.bfloat16)
    wt = W.T.astype(jnp.bfloat16)
    ball = (Bm.transpose(0, 2, 1).reshape(_ER, O) * _SCALING).astype(jnp.bfloat16)
    wb = jnp.concatenate([wt, ball], axis=0)
    b2 = b.reshape(1, O)
    out = pl.pallas_call(
        _fused_body,
        grid=(G,),
        in_specs=[
            pl.BlockSpec((1, 1, _TB), lambda i: (i, 0, 0)),
            pl.BlockSpec((_TB, D), lambda i: (i, 0)),
            pl.BlockSpec((D, _ER), lambda i: (0, 0)),
            pl.BlockSpec((D + _ER, O), lambda i: (0, 0)),
            pl.BlockSpec((1, O), lambda i: (0, 0)),
        ],
        out_specs=pl.BlockSpec((_TB, O), lambda i: (i, 0)),
        out_shape=jax.ShapeDtypeStruct((N, O), jnp.float32),
    )(role3, x_flat, at, wb, b2)
    return out.reshape(Bsz, T, O)
